# baseline (speedup 1.0000x reference)
import functools

import jax
import jax.numpy as jnp
from jax import lax
from jax.experimental import pallas as pl
from jax.experimental.pallas import tpu as pltpu
from jax.experimental.pallas import tpu_sc as plsc

NC = 2
NS = 16
LANES = 16
K = 80
RING = 4
GB = 8
LA = 2


def _sc_geometry(N: int, E: int):
    NW = NC * NS
    nch_real = -(-E // (NW * K))
    NCH = -(-nch_real // GB) * GB
    E_pad = NW * NCH * K
    lcm = 80 if K % 16 == 0 else K * 2
    N_pad = -(-(N + 128) // lcm) * lcm
    return NW, NCH, E_pad, N_pad


@functools.lru_cache(maxsize=None)
def _make_sc_agg(N: int, D: int, E: int, compute_deg: bool):
    NW, NCH, E_pad, N_pad = _sc_geometry(N, E)
    ngroups = NCH // GB
    nchunks = NCH
    ZR = K
    assert N_pad % ZR == 0
    nzch = N_pad // ZR
    zrounds = -(-nzch // NS)
    assert N_pad % 5 == 0 and (N_pad // 5) % 8 == 0
    DSL = N_pad // 5

    mesh = plsc.VectorSubcoreMesh(core_axis_name="c", subcore_axis_name="s")

    out_type = [jax.ShapeDtypeStruct((NC, N_pad, D), jnp.float32)]
    scratch = [
        pltpu.VMEM_SHARED((N_pad, D), jnp.float32),
        pltpu.VMEM((2, GB, K), jnp.int32),
        pltpu.VMEM((2, GB, K), jnp.int32),
        pltpu.VMEM((RING, K, D), jnp.float32),
    ]
    if compute_deg:
        out_type.append(jax.ShapeDtypeStruct((NC * N_pad,), jnp.float32))
        scratch += [
            pltpu.VMEM_SHARED((N_pad,), jnp.float32),
            pltpu.VMEM((K,), jnp.float32),
            pltpu.VMEM((DSL,), jnp.float32),
        ]
    scratch += [pltpu.SemaphoreType.DMA] * (RING * (3 if compute_deg else 2))

    @functools.partial(pl.kernel, out_type=tuple(out_type), mesh=mesh,
                       scratch_types=scratch)
    def sc_agg(x_hbm, src_hbm, dst_hbm, agg_out, *rest):
        if compute_deg:
            (deg_out, agg_sh, srcg, dstg, rowsb, deg_sh, onesb, zd, *sems) = rest
            gsem, ssem, dsem = sems[:RING], sems[RING:2 * RING], sems[2 * RING:]
        else:
            (agg_sh, srcg, dstg, rowsb, *sems) = rest
            gsem, ssem = sems[:RING], sems[RING:]
        zb = rowsb.at[0]
        c = lax.axis_index("c")
        s = lax.axis_index("s")
        wid = s * NC + c

        zvec = jnp.zeros((LANES,), jnp.float32)
        dlanes = D // LANES

        def zb_body(i, carry):
            r = i // dlanes
            col = (i % dlanes) * LANES
            zb[r, pl.ds(col, LANES)] = zvec
            return carry
        lax.fori_loop(0, ZR * dlanes, zb_body, 0)

        if compute_deg:
            ovec = jnp.ones((LANES,), jnp.float32)

            def zd_body(i, carry):
                zd[pl.ds(i * LANES, LANES)] = zvec
                return carry
            lax.fori_loop(0, DSL // LANES, zd_body, 0)

            def ones_body(i, carry):
                onesb[pl.ds(i * LANES, LANES)] = ovec
                return carry
            lax.fori_loop(0, K // LANES, ones_body, 0)
            if K % LANES:
                onesb[pl.ds(K - LANES, LANES)] = ovec

        def zcopy_body(k, carry):
            ch = s + k * NS

            @pl.when(ch < nzch)
            def _():
                pltpu.sync_copy(zb, agg_sh.at[pl.ds(ch * ZR, ZR)])
            return carry
        lax.fori_loop(0, zrounds, zcopy_body, 0)

        if compute_deg:
            @pl.when(s < 5)
            def _zero_deg():
                pltpu.sync_copy(zd, deg_sh.at[pl.ds(s * DSL, DSL)])

        plsc.subcore_barrier()

        def load_group(g_next, slot):
            pltpu.sync_copy(src_hbm.at[wid, pl.ds(g_next * GB, GB)],
                            srcg.at[slot])
            pltpu.sync_copy(dst_hbm.at[wid, pl.ds(g_next * GB, GB)],
                            dstg.at[slot])

        def chunk_work(g, u):
            j = g * GB + u
            b = u % RING
            sw = (u + LA) % RING
            p = lax.rem(g, 2)
            pltpu.make_async_copy(x_hbm.at[srcg.at[p, u]], rowsb.at[b],
                                  gsem[b]).wait()
            pltpu.async_copy(rowsb.at[b], agg_sh.at[dstg.at[p, u]], ssem[b],
                             add=True)
            if compute_deg:
                pltpu.async_copy(onesb, deg_sh.at[dstg.at[p, u]], dsem[b],
                                 add=True)

            @pl.when(j >= LA)
            def _drain_prev():
                pltpu.make_async_copy(rowsb.at[sw], agg_sh.at[dstg.at[p, u]],
                                      ssem[sw]).wait()
                if compute_deg:
                    pltpu.make_async_copy(onesb, deg_sh.at[dstg.at[p, u]],
                                          dsem[sw]).wait()

            @pl.when(j + LA < nchunks)
            def _fire_next():
                u2 = (u + LA) % GB
                p2 = lax.rem(g + (1 if u + LA >= GB else 0), 2)
                pltpu.async_copy(x_hbm.at[srcg.at[p2, u2]], rowsb.at[sw],
                                 gsem[sw])

        load_group(0, 0)
        for j0 in range(LA):
            pltpu.async_copy(x_hbm.at[srcg.at[0, j0]], rowsb.at[j0 % RING],
                             gsem[j0 % RING])

        def group(g, carry):
            chunk_work(g, 0)
            chunk_work(g, 1)

            @pl.when(g < ngroups - 1)
            def _prefetch():
                load_group(g + 1, lax.rem(g + 1, 2))
            for u in range(2, GB):
                chunk_work(g, u)
            return carry
        lax.fori_loop(0, ngroups, group, 0)

        for j in range(nchunks - LA, nchunks):
            b = (j % GB) % RING
            pltpu.make_async_copy(rowsb.at[b], agg_sh.at[dstg.at[0, 0]],
                                  ssem[b]).wait()
            if compute_deg:
                pltpu.make_async_copy(onesb, deg_sh.at[dstg.at[0, 0]],
                                      dsem[b]).wait()

        plsc.subcore_barrier()

        def ocopy_body(k, carry):
            ch = s + k * NS

            @pl.when(ch < nzch)
            def _():
                pltpu.sync_copy(agg_sh.at[pl.ds(ch * ZR, ZR)], zb)
                pltpu.sync_copy(zb, agg_out.at[c, pl.ds(ch * ZR, ZR)])
            return carry
        lax.fori_loop(0, zrounds, ocopy_body, 0)

        if compute_deg:
            @pl.when(s < 5)
            def _deg_out():
                pltpu.sync_copy(deg_sh.at[pl.ds(s * DSL, DSL)], zd)
                pltpu.sync_copy(zd, deg_out.at[pl.ds(c * N_pad + s * DSL, DSL)])

    return sc_agg


@functools.lru_cache(maxsize=None)
def _make_tc_lin(N: int, D: int, BN: int):
    grid = (N // BN,)

    def body(x, w, b, out):
        out[...] = jnp.dot(x[...], w[...],
                           preferred_element_type=jnp.float32) + b[...]

    row = pl.BlockSpec((BN, D), lambda i: (i, 0))
    return pl.pallas_call(
        body,
        grid=grid,
        in_specs=[row, pl.BlockSpec((D, D), lambda i: (0, 0)),
                  pl.BlockSpec((1, D), lambda i: (0, 0))],
        out_specs=row,
        out_shape=jax.ShapeDtypeStruct((N, D), jnp.float32),
    )


@functools.lru_cache(maxsize=None)
def _make_tc1(N: int, N_pad: int, D: int, BN: int):
    grid = (N // BN,)

    def body(agg0, agg1, deg0, deg1, x, wl, wr, b, out):
        deg = jnp.maximum(deg0[0] + deg1[0], 1.0)
        mean = (agg0[0] + agg1[0]) / deg
        acc = jnp.dot(mean, wl[...], preferred_element_type=jnp.float32)
        acc += jnp.dot(x[...], wr[...], preferred_element_type=jnp.float32)
        out[...] = jnp.maximum(acc + b[...], 0.0)

    agg0s = pl.BlockSpec((1, BN, D), lambda i: (0, i, 0))
    agg1s = pl.BlockSpec((1, BN, D), lambda i: (1, i, 0))
    deg0s = pl.BlockSpec((1, BN, 1), lambda i: (0, i, 0))
    deg1s = pl.BlockSpec((1, BN, 1), lambda i: (1, i, 0))
    row = pl.BlockSpec((BN, D), lambda i: (i, 0))
    full = pl.BlockSpec((D, D), lambda i: (0, 0))
    bias = pl.BlockSpec((1, D), lambda i: (0, 0))
    return pl.pallas_call(
        body,
        grid=grid,
        in_specs=[agg0s, agg1s, deg0s, deg1s, row, full, full, bias],
        out_specs=row,
        out_shape=jax.ShapeDtypeStruct((N, D), jnp.float32),
    )


@functools.lru_cache(maxsize=None)
def _make_tc2(N: int, N_pad: int, D: int, C1: int, C2: int, BN: int):
    grid = (N // BN,)

    def body(agg0, agg1, deg0, deg1, h, wl, wr, b, wh1, bh1, wh2, bh2,
             h2_out, o1_out, o2_out):
        deg = jnp.maximum(deg0[0] + deg1[0], 1.0)
        mean = (agg0[0] + agg1[0]) / deg
        acc = jnp.dot(mean, wl[...], preferred_element_type=jnp.float32)
        acc += jnp.dot(h[...], wr[...], preferred_element_type=jnp.float32)
        h2 = acc + b[...]
        h2_out[...] = h2
        o1_out[...] = jnp.dot(h2, wh1[...], preferred_element_type=jnp.float32) + bh1[...]
        o2_out[...] = jnp.dot(h2, wh2[...], preferred_element_type=jnp.float32) + bh2[...]

    agg0s = pl.BlockSpec((1, BN, D), lambda i: (0, i, 0))
    agg1s = pl.BlockSpec((1, BN, D), lambda i: (1, i, 0))
    deg0s = pl.BlockSpec((1, BN, 1), lambda i: (0, i, 0))
    deg1s = pl.BlockSpec((1, BN, 1), lambda i: (1, i, 0))
    row = pl.BlockSpec((BN, D), lambda i: (i, 0))
    full = pl.BlockSpec((D, D), lambda i: (0, 0))
    return pl.pallas_call(
        body,
        grid=grid,
        in_specs=[agg0s, agg1s, deg0s, deg1s, row, full, full,
                  pl.BlockSpec((1, D), lambda i: (0, 0)),
                  pl.BlockSpec((D, C1), lambda i: (0, 0)),
                  pl.BlockSpec((1, C1), lambda i: (0, 0)),
                  pl.BlockSpec((D, C2), lambda i: (0, 0)),
                  pl.BlockSpec((1, C2), lambda i: (0, 0))],
        out_specs=[row,
                   pl.BlockSpec((BN, C1), lambda i: (i, 0)),
                   pl.BlockSpec((BN, C2), lambda i: (i, 0))],
        out_shape=[jax.ShapeDtypeStruct((N, D), jnp.float32),
                   jax.ShapeDtypeStruct((N, C1), jnp.float32),
                   jax.ShapeDtypeStruct((N, C2), jnp.float32)],
    )


def kernel(x, edge_index, W1l, b1l, W1r, W2l, b2l, W2r, Wh1, bh1, Wh2, bh2):
    N, D = x.shape
    E = edge_index.shape[1]
    C1 = Wh1.shape[0]
    C2 = Wh2.shape[0]
    BN = 2000 if N % 2000 == 0 else 8
    NW, NCH, E_pad, N_pad = _sc_geometry(N, E)

    ei = edge_index.astype(jnp.int32)
    npad = E_pad - E
    pad_src = (jnp.arange(npad, dtype=jnp.int32) * 37) % N
    pad_dst = N + (jnp.arange(npad, dtype=jnp.int32) % 128)
    src3 = jnp.concatenate([ei[0], pad_src]).reshape(NW, NCH, K)
    dst3 = jnp.concatenate([ei[1], pad_dst]).reshape(NW, NCH, K)

    tc1 = _make_tc1(N, N_pad, D, BN)
    tc2 = _make_tc2(N, N_pad, D, C1, C2, BN)

    aggp, degp = _make_sc_agg(N, D, E, True)(x, src3, dst3)
    degp3 = degp.reshape(NC, N_pad, 1)
    h = tc1(aggp, aggp, degp3, degp3, x, W1l.T, W1r.T, b1l.reshape(1, D))

    (agg2p,) = _make_sc_agg(N, D, E, False)(h, src3, dst3)
    h2, out1, out2 = tc2(agg2p, agg2p, degp3, degp3, h, W2l.T, W2r.T,
                         b2l.reshape(1, D),
                         Wh1.T, bh1.reshape(1, C1), Wh2.T, bh2.reshape(1, C2))
    return (out1, out2, h2)


# speedup vs baseline: 1.0269x; 1.0269x over previous
import functools

import jax
import jax.numpy as jnp
from jax import lax
from jax.experimental import pallas as pl
from jax.experimental.pallas import tpu as pltpu
from jax.experimental.pallas import tpu_sc as plsc

NC = 2
NS = 16
LANES = 16
K = 80
RING = 4
GB = 8
LA = 2


def _sc_geometry(N: int, E: int):
    NW = NC * NS
    nch_real = -(-E // (NW * K))
    NCH = -(-nch_real // GB) * GB
    E_pad = NW * NCH * K
    lcm = 80 if K % 16 == 0 else K * 2
    N_pad = -(-(N + 128) // lcm) * lcm
    return NW, NCH, E_pad, N_pad


@functools.lru_cache(maxsize=None)
def _make_sc_agg(N: int, D: int, E: int, compute_deg: bool):
    NW, NCH, E_pad, N_pad = _sc_geometry(N, E)
    ngroups = NCH // GB
    nchunks = NCH
    ZR = K
    assert N_pad % ZR == 0
    nzch = N_pad // ZR
    zrounds = -(-nzch // NS)
    assert N_pad % 5 == 0 and (N_pad // 5) % 8 == 0
    DSL = N_pad // 5

    mesh = plsc.VectorSubcoreMesh(core_axis_name="c", subcore_axis_name="s")

    out_type = [jax.ShapeDtypeStruct((NC, N_pad, D), jnp.float32)]
    scratch = [
        pltpu.VMEM_SHARED((N_pad, D), jnp.float32),
        pltpu.VMEM((2, GB, K), jnp.int32),
        pltpu.VMEM((2, GB, K), jnp.int32),
        pltpu.VMEM((RING, K, D), jnp.float32),
    ]
    if compute_deg:
        out_type.append(jax.ShapeDtypeStruct((NC * N_pad,), jnp.float32))
        scratch += [
            pltpu.VMEM_SHARED((N_pad,), jnp.float32),
            pltpu.VMEM((K,), jnp.float32),
            pltpu.VMEM((DSL,), jnp.float32),
        ]
    scratch += [pltpu.SemaphoreType.DMA] * (RING * (3 if compute_deg else 2))

    @functools.partial(pl.kernel, out_type=tuple(out_type), mesh=mesh,
                       scratch_types=scratch)
    def sc_agg(x_hbm, src_hbm, dst_hbm, agg_out, *rest):
        if compute_deg:
            (deg_out, agg_sh, srcg, dstg, rowsb, deg_sh, onesb, zd, *sems) = rest
            gsem, ssem, dsem = sems[:RING], sems[RING:2 * RING], sems[2 * RING:]
        else:
            (agg_sh, srcg, dstg, rowsb, *sems) = rest
            gsem, ssem = sems[:RING], sems[RING:]
        zb = rowsb.at[RING - 1]
        c = lax.axis_index("c")
        s = lax.axis_index("s")
        wid = s * NC + c

        pltpu.sync_copy(src_hbm.at[wid, pl.ds(0, GB)], srcg.at[0])
        pltpu.sync_copy(dst_hbm.at[wid, pl.ds(0, GB)], dstg.at[0])
        for j0 in range(LA):
            pltpu.async_copy(x_hbm.at[srcg.at[0, j0]], rowsb.at[j0 % RING],
                             gsem[j0 % RING])

        zvec = jnp.zeros((LANES,), jnp.float32)
        dlanes = D // LANES

        def zb_body(i, carry):
            r = i // dlanes
            col = (i % dlanes) * LANES
            zb[r, pl.ds(col, LANES)] = zvec
            return carry
        lax.fori_loop(0, ZR * dlanes, zb_body, 0)

        if compute_deg:
            ovec = jnp.ones((LANES,), jnp.float32)

            def zd_body(i, carry):
                zd[pl.ds(i * LANES, LANES)] = zvec
                return carry
            lax.fori_loop(0, DSL // LANES, zd_body, 0)

            def ones_body(i, carry):
                onesb[pl.ds(i * LANES, LANES)] = ovec
                return carry
            lax.fori_loop(0, K // LANES, ones_body, 0)
            if K % LANES:
                onesb[pl.ds(K - LANES, LANES)] = ovec

        for k in range(zrounds):
            ch = s + k * NS

            @pl.when(ch < nzch)
            def _():
                pltpu.async_copy(zb, agg_sh.at[pl.ds(ch * ZR, ZR)], ssem[0])

        if compute_deg:
            @pl.when(s < 5)
            def _zero_deg():
                pltpu.async_copy(zd, deg_sh.at[pl.ds(s * DSL, DSL)], ssem[1])

        for k in range(zrounds):
            ch = s + k * NS

            @pl.when(ch < nzch)
            def _():
                pltpu.make_async_copy(zb, agg_sh.at[pl.ds(ch * ZR, ZR)],
                                      ssem[0]).wait()

        if compute_deg:
            @pl.when(s < 5)
            def _wait_zero_deg():
                pltpu.make_async_copy(zd, deg_sh.at[pl.ds(s * DSL, DSL)],
                                      ssem[1]).wait()

        plsc.subcore_barrier()

        def load_group(g_next, slot):
            pltpu.sync_copy(src_hbm.at[wid, pl.ds(g_next * GB, GB)],
                            srcg.at[slot])
            pltpu.sync_copy(dst_hbm.at[wid, pl.ds(g_next * GB, GB)],
                            dstg.at[slot])

        def chunk_work(g, u):
            j = g * GB + u
            b = u % RING
            sw = (u + LA) % RING
            p = lax.rem(g, 2)
            pltpu.make_async_copy(x_hbm.at[srcg.at[p, u]], rowsb.at[b],
                                  gsem[b]).wait()
            pltpu.async_copy(rowsb.at[b], agg_sh.at[dstg.at[p, u]], ssem[b],
                             add=True)
            if compute_deg:
                pltpu.async_copy(onesb, deg_sh.at[dstg.at[p, u]], dsem[b],
                                 add=True)

            @pl.when(j >= LA)
            def _drain_prev():
                pltpu.make_async_copy(rowsb.at[sw], agg_sh.at[dstg.at[p, u]],
                                      ssem[sw]).wait()
                if compute_deg:
                    pltpu.make_async_copy(onesb, deg_sh.at[dstg.at[p, u]],
                                          dsem[sw]).wait()

            @pl.when(j + LA < nchunks)
            def _fire_next():
                u2 = (u + LA) % GB
                p2 = lax.rem(g + (1 if u + LA >= GB else 0), 2)
                pltpu.async_copy(x_hbm.at[srcg.at[p2, u2]], rowsb.at[sw],
                                 gsem[sw])

        def group(g, carry):
            chunk_work(g, 0)
            chunk_work(g, 1)

            @pl.when(g < ngroups - 1)
            def _prefetch():
                load_group(g + 1, lax.rem(g + 1, 2))
            for u in range(2, GB):
                chunk_work(g, u)
            return carry
        lax.fori_loop(0, ngroups, group, 0)

        for j in range(nchunks - LA, nchunks):
            b = (j % GB) % RING
            pltpu.make_async_copy(rowsb.at[b], agg_sh.at[dstg.at[0, 0]],
                                  ssem[b]).wait()
            if compute_deg:
                pltpu.make_async_copy(onesb, deg_sh.at[dstg.at[0, 0]],
                                      dsem[b]).wait()

        plsc.subcore_barrier()

        if compute_deg:
            @pl.when(s < 5)
            def _deg_read():
                pltpu.sync_copy(deg_sh.at[pl.ds(s * DSL, DSL)], zd)
                pltpu.async_copy(zd, deg_out.at[pl.ds(c * N_pad + s * DSL, DSL)],
                                 dsem[0])

        for k in range(zrounds):
            ch = s + k * NS
            slot = k % 2
            if k >= 2:
                chp = s + (k - 2) * NS

                @pl.when(chp < nzch)
                def _wait_prev():
                    pltpu.make_async_copy(rowsb.at[slot],
                                          agg_out.at[c, pl.ds(chp * ZR, ZR)],
                                          ssem[slot]).wait()

            @pl.when(ch < nzch)
            def _stage():
                pltpu.async_copy(agg_sh.at[pl.ds(ch * ZR, ZR)], rowsb.at[slot],
                                 gsem[slot]).wait()
                pltpu.async_copy(rowsb.at[slot],
                                 agg_out.at[c, pl.ds(ch * ZR, ZR)], ssem[slot])

        for k in range(max(zrounds - 2, 0), zrounds):
            chl = s + k * NS
            slot = k % 2

            @pl.when(chl < nzch)
            def _drain_out():
                pltpu.make_async_copy(rowsb.at[slot],
                                      agg_out.at[c, pl.ds(chl * ZR, ZR)],
                                      ssem[slot]).wait()

        if compute_deg:
            @pl.when(s < 5)
            def _deg_drain():
                pltpu.make_async_copy(zd,
                                      deg_out.at[pl.ds(c * N_pad + s * DSL, DSL)],
                                      dsem[0]).wait()

    return sc_agg


@functools.lru_cache(maxsize=None)
def _make_tc_lin(N: int, D: int, BN: int):
    grid = (N // BN,)

    def body(x, w, b, out):
        out[...] = jnp.dot(x[...], w[...],
                           preferred_element_type=jnp.float32) + b[...]

    row = pl.BlockSpec((BN, D), lambda i: (i, 0))
    return pl.pallas_call(
        body,
        grid=grid,
        in_specs=[row, pl.BlockSpec((D, D), lambda i: (0, 0)),
                  pl.BlockSpec((1, D), lambda i: (0, 0))],
        out_specs=row,
        out_shape=jax.ShapeDtypeStruct((N, D), jnp.float32),
    )


@functools.lru_cache(maxsize=None)
def _make_tc1(N: int, N_pad: int, D: int, BN: int):
    grid = (N // BN,)

    def body(agg0, agg1, deg0, deg1, x, wl, wr, b, out):
        deg = jnp.maximum(deg0[0] + deg1[0], 1.0)
        mean = (agg0[0] + agg1[0]) / deg
        acc = jnp.dot(mean, wl[...], preferred_element_type=jnp.float32)
        acc += jnp.dot(x[...], wr[...], preferred_element_type=jnp.float32)
        out[...] = jnp.maximum(acc + b[...], 0.0)

    agg0s = pl.BlockSpec((1, BN, D), lambda i: (0, i, 0))
    agg1s = pl.BlockSpec((1, BN, D), lambda i: (1, i, 0))
    deg0s = pl.BlockSpec((1, BN, 1), lambda i: (0, i, 0))
    deg1s = pl.BlockSpec((1, BN, 1), lambda i: (1, i, 0))
    row = pl.BlockSpec((BN, D), lambda i: (i, 0))
    full = pl.BlockSpec((D, D), lambda i: (0, 0))
    bias = pl.BlockSpec((1, D), lambda i: (0, 0))
    return pl.pallas_call(
        body,
        grid=grid,
        in_specs=[agg0s, agg1s, deg0s, deg1s, row, full, full, bias],
        out_specs=row,
        out_shape=jax.ShapeDtypeStruct((N, D), jnp.float32),
    )


@functools.lru_cache(maxsize=None)
def _make_tc2(N: int, N_pad: int, D: int, C1: int, C2: int, BN: int):
    grid = (N // BN,)

    def body(agg0, agg1, deg0, deg1, h, wl, wr, b, wh1, bh1, wh2, bh2,
             h2_out, o1_out, o2_out):
        deg = jnp.maximum(deg0[0] + deg1[0], 1.0)
        mean = (agg0[0] + agg1[0]) / deg
        acc = jnp.dot(mean, wl[...], preferred_element_type=jnp.float32)
        acc += jnp.dot(h[...], wr[...], preferred_element_type=jnp.float32)
        h2 = acc + b[...]
        h2_out[...] = h2
        o1_out[...] = jnp.dot(h2, wh1[...], preferred_element_type=jnp.float32) + bh1[...]
        o2_out[...] = jnp.dot(h2, wh2[...], preferred_element_type=jnp.float32) + bh2[...]

    agg0s = pl.BlockSpec((1, BN, D), lambda i: (0, i, 0))
    agg1s = pl.BlockSpec((1, BN, D), lambda i: (1, i, 0))
    deg0s = pl.BlockSpec((1, BN, 1), lambda i: (0, i, 0))
    deg1s = pl.BlockSpec((1, BN, 1), lambda i: (1, i, 0))
    row = pl.BlockSpec((BN, D), lambda i: (i, 0))
    full = pl.BlockSpec((D, D), lambda i: (0, 0))
    return pl.pallas_call(
        body,
        grid=grid,
        in_specs=[agg0s, agg1s, deg0s, deg1s, row, full, full,
                  pl.BlockSpec((1, D), lambda i: (0, 0)),
                  pl.BlockSpec((D, C1), lambda i: (0, 0)),
                  pl.BlockSpec((1, C1), lambda i: (0, 0)),
                  pl.BlockSpec((D, C2), lambda i: (0, 0)),
                  pl.BlockSpec((1, C2), lambda i: (0, 0))],
        out_specs=[row,
                   pl.BlockSpec((BN, C1), lambda i: (i, 0)),
                   pl.BlockSpec((BN, C2), lambda i: (i, 0))],
        out_shape=[jax.ShapeDtypeStruct((N, D), jnp.float32),
                   jax.ShapeDtypeStruct((N, C1), jnp.float32),
                   jax.ShapeDtypeStruct((N, C2), jnp.float32)],
    )


def kernel(x, edge_index, W1l, b1l, W1r, W2l, b2l, W2r, Wh1, bh1, Wh2, bh2):
    N, D = x.shape
    E = edge_index.shape[1]
    C1 = Wh1.shape[0]
    C2 = Wh2.shape[0]
    BN = 2000 if N % 2000 == 0 else 8
    NW, NCH, E_pad, N_pad = _sc_geometry(N, E)

    ei = edge_index.astype(jnp.int32)
    npad = E_pad - E
    pad_src = (jnp.arange(npad, dtype=jnp.int32) * 37) % N
    pad_dst = N + (jnp.arange(npad, dtype=jnp.int32) % 128)
    src3 = jnp.concatenate([ei[0], pad_src]).reshape(NW, NCH, K)
    dst3 = jnp.concatenate([ei[1], pad_dst]).reshape(NW, NCH, K)

    tc1 = _make_tc1(N, N_pad, D, BN)
    tc2 = _make_tc2(N, N_pad, D, C1, C2, BN)

    aggp, degp = _make_sc_agg(N, D, E, True)(x, src3, dst3)
    degp3 = degp.reshape(NC, N_pad, 1)
    h = tc1(aggp, aggp, degp3, degp3, x, W1l.T, W1r.T, b1l.reshape(1, D))

    (agg2p,) = _make_sc_agg(N, D, E, False)(h, src3, dst3)
    h2, out1, out2 = tc2(agg2p, agg2p, degp3, degp3, h, W2l.T, W2r.T,
                         b2l.reshape(1, D),
                         Wh1.T, bh1.reshape(1, C1), Wh2.T, bh2.reshape(1, C2))
    return (out1, out2, h2)


# speedup vs baseline: 1.0305x; 1.0035x over previous
import functools

import jax
import jax.numpy as jnp
from jax import lax
from jax.experimental import pallas as pl
from jax.experimental.pallas import tpu as pltpu
from jax.experimental.pallas import tpu_sc as plsc

NC = 2
NS = 16
LANES = 16
K = 80
RING = 4
GBMAX = 16
LA = 2


def _sc_geometry(N: int, E: int):
    NW = NC * NS
    nch_real = -(-E // (NW * K))
    NCH = -(-nch_real // GBMAX) * GBMAX
    E_pad = NW * NCH * K
    lcm = 80 if K % 16 == 0 else K * 2
    N_pad = -(-(N + 128) // lcm) * lcm
    return NW, NCH, E_pad, N_pad


@functools.lru_cache(maxsize=None)
def _make_sc_agg(N: int, D: int, E: int, compute_deg: bool, GB: int):
    NW, NCH, E_pad, N_pad = _sc_geometry(N, E)
    assert NCH % GB == 0
    ngroups = NCH // GB
    nchunks = NCH
    ZR = K
    assert N_pad % ZR == 0
    nzch = N_pad // ZR
    zrounds = -(-nzch // NS)
    assert N_pad % 10 == 0 and (N_pad // 10) % 8 == 0
    DSL = N_pad // 10

    mesh = plsc.VectorSubcoreMesh(core_axis_name="c", subcore_axis_name="s")

    out_type = [jax.ShapeDtypeStruct((NC, N_pad, D), jnp.float32)]
    scratch = [
        pltpu.VMEM_SHARED((N_pad, D), jnp.float32),
        pltpu.VMEM((2, GB, K), jnp.int32),
        pltpu.VMEM((2, GB, K), jnp.int32),
        pltpu.VMEM((RING, K, D), jnp.float32),
    ]
    if compute_deg:
        out_type.append(jax.ShapeDtypeStruct((NC * N_pad,), jnp.float32))
        scratch += [
            pltpu.VMEM_SHARED((N_pad,), jnp.float32),
            pltpu.VMEM((K,), jnp.float32),
            pltpu.VMEM((DSL,), jnp.float32),
        ]
    scratch += [pltpu.SemaphoreType.DMA] * (RING * (3 if compute_deg else 2))

    @functools.partial(pl.kernel, out_type=tuple(out_type), mesh=mesh,
                       scratch_types=scratch)
    def sc_agg(x_hbm, src_hbm, dst_hbm, agg_out, *rest):
        if compute_deg:
            (deg_out, agg_sh, srcg, dstg, rowsb, deg_sh, onesb, zd, *sems) = rest
            gsem, ssem, dsem = sems[:RING], sems[RING:2 * RING], sems[2 * RING:]
        else:
            (agg_sh, srcg, dstg, rowsb, *sems) = rest
            gsem, ssem = sems[:RING], sems[RING:]
        zb = rowsb.at[RING - 1]
        c = lax.axis_index("c")
        s = lax.axis_index("s")
        wid = s * NC + c

        pltpu.sync_copy(src_hbm.at[wid, pl.ds(0, GB)], srcg.at[0])
        pltpu.sync_copy(dst_hbm.at[wid, pl.ds(0, GB)], dstg.at[0])
        for j0 in range(LA):
            pltpu.async_copy(x_hbm.at[srcg.at[0, j0]], rowsb.at[j0 % RING],
                             gsem[j0 % RING])

        zvec = jnp.zeros((LANES,), jnp.float32)
        dlanes = D // LANES

        def zb_body(i, carry):
            r = i // dlanes
            col = (i % dlanes) * LANES
            zb[r, pl.ds(col, LANES)] = zvec
            return carry
        lax.fori_loop(0, ZR * dlanes, zb_body, 0)

        if compute_deg:
            ovec = jnp.ones((LANES,), jnp.float32)

            def zd_body(i, carry):
                zd[pl.ds(i * LANES, LANES)] = zvec
                return carry
            lax.fori_loop(0, DSL // LANES, zd_body, 0)
            if DSL % LANES:
                zd[pl.ds(DSL - LANES, LANES)] = zvec

            def ones_body(i, carry):
                onesb[pl.ds(i * LANES, LANES)] = ovec
                return carry
            lax.fori_loop(0, K // LANES, ones_body, 0)
            if K % LANES:
                onesb[pl.ds(K - LANES, LANES)] = ovec

        for k in range(zrounds):
            ch = s + k * NS

            @pl.when(ch < nzch)
            def _():
                pltpu.async_copy(zb, agg_sh.at[pl.ds(ch * ZR, ZR)], ssem[0])

        if compute_deg:
            @pl.when(s < 10)
            def _zero_deg():
                pltpu.async_copy(zd, deg_sh.at[pl.ds(s * DSL, DSL)], ssem[1])

        for k in range(zrounds):
            ch = s + k * NS

            @pl.when(ch < nzch)
            def _():
                pltpu.make_async_copy(zb, agg_sh.at[pl.ds(ch * ZR, ZR)],
                                      ssem[0]).wait()

        if compute_deg:
            @pl.when(s < 10)
            def _wait_zero_deg():
                pltpu.make_async_copy(zd, deg_sh.at[pl.ds(s * DSL, DSL)],
                                      ssem[1]).wait()

        plsc.subcore_barrier()

        def load_group(g_next, slot):
            pltpu.sync_copy(src_hbm.at[wid, pl.ds(g_next * GB, GB)],
                            srcg.at[slot])
            pltpu.sync_copy(dst_hbm.at[wid, pl.ds(g_next * GB, GB)],
                            dstg.at[slot])

        def chunk_work(g, u):
            j = g * GB + u
            b = u % RING
            sw = (u + LA) % RING
            p = lax.rem(g, 2)
            pltpu.make_async_copy(x_hbm.at[srcg.at[p, u]], rowsb.at[b],
                                  gsem[b]).wait()
            pltpu.async_copy(rowsb.at[b], agg_sh.at[dstg.at[p, u]], ssem[b],
                             add=True)
            if compute_deg:
                pltpu.async_copy(onesb, deg_sh.at[dstg.at[p, u]], dsem[b],
                                 add=True)

            @pl.when(j >= LA)
            def _drain_prev():
                pltpu.make_async_copy(rowsb.at[sw], agg_sh.at[dstg.at[p, u]],
                                      ssem[sw]).wait()
                if compute_deg:
                    pltpu.make_async_copy(onesb, deg_sh.at[dstg.at[p, u]],
                                          dsem[sw]).wait()

            @pl.when(j + LA < nchunks)
            def _fire_next():
                u2 = (u + LA) % GB
                p2 = lax.rem(g + (1 if u + LA >= GB else 0), 2)
                pltpu.async_copy(x_hbm.at[srcg.at[p2, u2]], rowsb.at[sw],
                                 gsem[sw])

        def group(g, carry):
            chunk_work(g, 0)
            chunk_work(g, 1)

            @pl.when(g < ngroups - 1)
            def _prefetch():
                load_group(g + 1, lax.rem(g + 1, 2))
            for u in range(2, GB):
                chunk_work(g, u)
            return carry
        lax.fori_loop(0, ngroups, group, 0)

        for j in range(nchunks - LA, nchunks):
            b = (j % GB) % RING
            pltpu.make_async_copy(rowsb.at[b], agg_sh.at[dstg.at[0, 0]],
                                  ssem[b]).wait()
            if compute_deg:
                pltpu.make_async_copy(onesb, deg_sh.at[dstg.at[0, 0]],
                                      dsem[b]).wait()

        plsc.subcore_barrier()

        if compute_deg:
            @pl.when(s < 10)
            def _deg_read():
                pltpu.sync_copy(deg_sh.at[pl.ds(s * DSL, DSL)], zd)
                pltpu.async_copy(zd, deg_out.at[pl.ds(c * N_pad + s * DSL, DSL)],
                                 dsem[0])

        for k in range(zrounds):
            ch = s + k * NS
            slot = k % 2
            if k >= 2:
                chp = s + (k - 2) * NS

                @pl.when(chp < nzch)
                def _wait_prev():
                    pltpu.make_async_copy(rowsb.at[slot],
                                          agg_out.at[c, pl.ds(chp * ZR, ZR)],
                                          ssem[slot]).wait()

            @pl.when(ch < nzch)
            def _stage():
                pltpu.async_copy(agg_sh.at[pl.ds(ch * ZR, ZR)], rowsb.at[slot],
                                 gsem[slot]).wait()
                pltpu.async_copy(rowsb.at[slot],
                                 agg_out.at[c, pl.ds(ch * ZR, ZR)], ssem[slot])

        for k in range(max(zrounds - 2, 0), zrounds):
            chl = s + k * NS
            slot = k % 2

            @pl.when(chl < nzch)
            def _drain_out():
                pltpu.make_async_copy(rowsb.at[slot],
                                      agg_out.at[c, pl.ds(chl * ZR, ZR)],
                                      ssem[slot]).wait()

        if compute_deg:
            @pl.when(s < 10)
            def _deg_drain():
                pltpu.make_async_copy(zd,
                                      deg_out.at[pl.ds(c * N_pad + s * DSL, DSL)],
                                      dsem[0]).wait()

    return sc_agg


@functools.lru_cache(maxsize=None)
def _make_tc_lin(N: int, D: int, BN: int):
    grid = (N // BN,)

    def body(x, w, b, out):
        out[...] = jnp.dot(x[...], w[...],
                           preferred_element_type=jnp.float32) + b[...]

    row = pl.BlockSpec((BN, D), lambda i: (i, 0))
    return pl.pallas_call(
        body,
        grid=grid,
        in_specs=[row, pl.BlockSpec((D, D), lambda i: (0, 0)),
                  pl.BlockSpec((1, D), lambda i: (0, 0))],
        out_specs=row,
        out_shape=jax.ShapeDtypeStruct((N, D), jnp.float32),
    )


@functools.lru_cache(maxsize=None)
def _make_tc1(N: int, N_pad: int, D: int, BN: int):
    grid = (N // BN,)

    def body(agg0, agg1, deg0, deg1, x, wl, wr, b, out):
        deg = jnp.maximum(deg0[0] + deg1[0], 1.0)
        mean = (agg0[0] + agg1[0]) / deg
        acc = jnp.dot(mean, wl[...], preferred_element_type=jnp.float32)
        acc += jnp.dot(x[...], wr[...], preferred_element_type=jnp.float32)
        out[...] = jnp.maximum(acc + b[...], 0.0)

    agg0s = pl.BlockSpec((1, BN, D), lambda i: (0, i, 0))
    agg1s = pl.BlockSpec((1, BN, D), lambda i: (1, i, 0))
    deg0s = pl.BlockSpec((1, BN, 1), lambda i: (0, i, 0))
    deg1s = pl.BlockSpec((1, BN, 1), lambda i: (1, i, 0))
    row = pl.BlockSpec((BN, D), lambda i: (i, 0))
    full = pl.BlockSpec((D, D), lambda i: (0, 0))
    bias = pl.BlockSpec((1, D), lambda i: (0, 0))
    return pl.pallas_call(
        body,
        grid=grid,
        in_specs=[agg0s, agg1s, deg0s, deg1s, row, full, full, bias],
        out_specs=row,
        out_shape=jax.ShapeDtypeStruct((N, D), jnp.float32),
    )


@functools.lru_cache(maxsize=None)
def _make_tc2(N: int, N_pad: int, D: int, C1: int, C2: int, BN: int):
    grid = (N // BN,)

    def body(agg0, agg1, deg0, deg1, h, wl, wr, b, wh1, bh1, wh2, bh2,
             h2_out, o1_out, o2_out):
        deg = jnp.maximum(deg0[0] + deg1[0], 1.0)
        mean = (agg0[0] + agg1[0]) / deg
        acc = jnp.dot(mean, wl[...], preferred_element_type=jnp.float32)
        acc += jnp.dot(h[...], wr[...], preferred_element_type=jnp.float32)
        h2 = acc + b[...]
        h2_out[...] = h2
        o1_out[...] = jnp.dot(h2, wh1[...], preferred_element_type=jnp.float32) + bh1[...]
        o2_out[...] = jnp.dot(h2, wh2[...], preferred_element_type=jnp.float32) + bh2[...]

    agg0s = pl.BlockSpec((1, BN, D), lambda i: (0, i, 0))
    agg1s = pl.BlockSpec((1, BN, D), lambda i: (1, i, 0))
    deg0s = pl.BlockSpec((1, BN, 1), lambda i: (0, i, 0))
    deg1s = pl.BlockSpec((1, BN, 1), lambda i: (1, i, 0))
    row = pl.BlockSpec((BN, D), lambda i: (i, 0))
    full = pl.BlockSpec((D, D), lambda i: (0, 0))
    return pl.pallas_call(
        body,
        grid=grid,
        in_specs=[agg0s, agg1s, deg0s, deg1s, row, full, full,
                  pl.BlockSpec((1, D), lambda i: (0, 0)),
                  pl.BlockSpec((D, C1), lambda i: (0, 0)),
                  pl.BlockSpec((1, C1), lambda i: (0, 0)),
                  pl.BlockSpec((D, C2), lambda i: (0, 0)),
                  pl.BlockSpec((1, C2), lambda i: (0, 0))],
        out_specs=[row,
                   pl.BlockSpec((BN, C1), lambda i: (i, 0)),
                   pl.BlockSpec((BN, C2), lambda i: (i, 0))],
        out_shape=[jax.ShapeDtypeStruct((N, D), jnp.float32),
                   jax.ShapeDtypeStruct((N, C1), jnp.float32),
                   jax.ShapeDtypeStruct((N, C2), jnp.float32)],
    )


def kernel(x, edge_index, W1l, b1l, W1r, W2l, b2l, W2r, Wh1, bh1, Wh2, bh2):
    N, D = x.shape
    E = edge_index.shape[1]
    C1 = Wh1.shape[0]
    C2 = Wh2.shape[0]
    BN = 2000 if N % 2000 == 0 else 8
    NW, NCH, E_pad, N_pad = _sc_geometry(N, E)

    ei = edge_index.astype(jnp.int32)
    npad = E_pad - E
    pad_src = (jnp.arange(npad, dtype=jnp.int32) * 37) % N
    pad_dst = N + (jnp.arange(npad, dtype=jnp.int32) % 128)
    src3 = jnp.concatenate([ei[0], pad_src]).reshape(NW, NCH, K)
    dst3 = jnp.concatenate([ei[1], pad_dst]).reshape(NW, NCH, K)

    tc1 = _make_tc1(N, N_pad, D, BN)
    tc2 = _make_tc2(N, N_pad, D, C1, C2, BN)

    aggp, degp = _make_sc_agg(N, D, E, True, 8)(x, src3, dst3)
    degp3 = degp.reshape(NC, N_pad, 1)
    h = tc1(aggp, aggp, degp3, degp3, x, W1l.T, W1r.T, b1l.reshape(1, D))

    (agg2p,) = _make_sc_agg(N, D, E, False, 16)(h, src3, dst3)
    h2, out1, out2 = tc2(agg2p, agg2p, degp3, degp3, h, W2l.T, W2r.T,
                         b2l.reshape(1, D),
                         Wh1.T, bh1.reshape(1, C1), Wh2.T, bh2.reshape(1, C2))
    return (out1, out2, h2)


# speedup vs baseline: 1.1082x; 1.0754x over previous
import functools

import jax
import jax.numpy as jnp
from jax import lax
from jax.experimental import pallas as pl
from jax.experimental.pallas import tpu as pltpu
from jax.experimental.pallas import tpu_sc as plsc

NC = 2
NS = 16
LANES = 16
K = 80
RING = 4
GBMAX = 16
LAG = 3
LAS = 1


def _sc_geometry(N: int, E: int):
    NW = NC * NS
    nch_real = -(-E // (NW * K))
    NCH = -(-nch_real // GBMAX) * GBMAX
    E_pad = NW * NCH * K
    lcm = 80 if K % 16 == 0 else K * 2
    N_pad = -(-(N + 128) // lcm) * lcm
    return NW, NCH, E_pad, N_pad


@functools.lru_cache(maxsize=None)
def _make_sc_agg(N: int, D: int, E: int, compute_deg: bool, GB: int):
    NW, NCH, E_pad, N_pad = _sc_geometry(N, E)
    assert NCH % GB == 0
    ngroups = NCH // GB
    nchunks = NCH
    ZR = K
    assert N_pad % ZR == 0
    nzch = N_pad // ZR
    zrounds = -(-nzch // NS)
    assert N_pad % 10 == 0 and (N_pad // 10) % 8 == 0
    DSL = N_pad // 10

    mesh = plsc.VectorSubcoreMesh(core_axis_name="c", subcore_axis_name="s")

    out_type = [jax.ShapeDtypeStruct((NC, N_pad, D), jnp.float32)]
    scratch = [
        pltpu.VMEM_SHARED((N_pad, D), jnp.float32),
        pltpu.VMEM((2, GB, K), jnp.int32),
        pltpu.VMEM((2, GB, K), jnp.int32),
        pltpu.VMEM((RING, K, D), jnp.float32),
    ]
    if compute_deg:
        out_type.append(jax.ShapeDtypeStruct((NC * N_pad,), jnp.float32))
        scratch += [
            pltpu.VMEM_SHARED((N_pad,), jnp.float32),
            pltpu.VMEM((K,), jnp.float32),
            pltpu.VMEM((DSL,), jnp.float32),
        ]
    scratch += [pltpu.SemaphoreType.DMA] * (RING * (3 if compute_deg else 2))

    @functools.partial(pl.kernel, out_type=tuple(out_type), mesh=mesh,
                       scratch_types=scratch)
    def sc_agg(x_hbm, src_hbm, dst_hbm, agg_out, *rest):
        if compute_deg:
            (deg_out, agg_sh, srcg, dstg, rowsb, deg_sh, onesb, zd, *sems) = rest
            gsem, ssem, dsem = sems[:RING], sems[RING:2 * RING], sems[2 * RING:]
        else:
            (agg_sh, srcg, dstg, rowsb, *sems) = rest
            gsem, ssem = sems[:RING], sems[RING:]
        zb = rowsb.at[RING - 1]
        c = lax.axis_index("c")
        s = lax.axis_index("s")
        wid = s * NC + c

        pltpu.sync_copy(src_hbm.at[wid, pl.ds(0, GB)], srcg.at[0])
        pltpu.sync_copy(dst_hbm.at[wid, pl.ds(0, GB)], dstg.at[0])
        for j0 in range(LAG):
            pltpu.async_copy(x_hbm.at[srcg.at[0, j0]], rowsb.at[j0 % RING],
                             gsem[j0 % RING])

        zvec = jnp.zeros((LANES,), jnp.float32)
        dlanes = D // LANES

        def zb_body(i, carry):
            r = i // dlanes
            col = (i % dlanes) * LANES
            zb[r, pl.ds(col, LANES)] = zvec
            return carry
        lax.fori_loop(0, ZR * dlanes, zb_body, 0)

        if compute_deg:
            ovec = jnp.ones((LANES,), jnp.float32)

            def zd_body(i, carry):
                zd[pl.ds(i * LANES, LANES)] = zvec
                return carry
            lax.fori_loop(0, DSL // LANES, zd_body, 0)
            if DSL % LANES:
                zd[pl.ds(DSL - LANES, LANES)] = zvec

            def ones_body(i, carry):
                onesb[pl.ds(i * LANES, LANES)] = ovec
                return carry
            lax.fori_loop(0, K // LANES, ones_body, 0)
            if K % LANES:
                onesb[pl.ds(K - LANES, LANES)] = ovec

        for k in range(zrounds):
            ch = s + k * NS

            @pl.when(ch < nzch)
            def _():
                pltpu.async_copy(zb, agg_sh.at[pl.ds(ch * ZR, ZR)], ssem[0])

        if compute_deg:
            @pl.when(s < 10)
            def _zero_deg():
                pltpu.async_copy(zd, deg_sh.at[pl.ds(s * DSL, DSL)], ssem[1])

        for k in range(zrounds):
            ch = s + k * NS

            @pl.when(ch < nzch)
            def _():
                pltpu.make_async_copy(zb, agg_sh.at[pl.ds(ch * ZR, ZR)],
                                      ssem[0]).wait()

        if compute_deg:
            @pl.when(s < 10)
            def _wait_zero_deg():
                pltpu.make_async_copy(zd, deg_sh.at[pl.ds(s * DSL, DSL)],
                                      ssem[1]).wait()

        plsc.subcore_barrier()

        def load_group(g_next, slot):
            pltpu.sync_copy(src_hbm.at[wid, pl.ds(g_next * GB, GB)],
                            srcg.at[slot])
            pltpu.sync_copy(dst_hbm.at[wid, pl.ds(g_next * GB, GB)],
                            dstg.at[slot])

        def chunk_work(g, u):
            j = g * GB + u
            b = u % RING
            sw = (u + LAG) % RING
            p = lax.rem(g, 2)
            pltpu.make_async_copy(x_hbm.at[srcg.at[p, u]], rowsb.at[b],
                                  gsem[b]).wait()
            pltpu.async_copy(rowsb.at[b], agg_sh.at[dstg.at[p, u]], ssem[b],
                             add=True)
            if compute_deg:
                pltpu.async_copy(onesb, deg_sh.at[dstg.at[p, u]], dsem[b],
                                 add=True)

            @pl.when(j >= LAS)
            def _drain_prev():
                pltpu.make_async_copy(rowsb.at[sw], agg_sh.at[dstg.at[p, u]],
                                      ssem[sw]).wait()
                if compute_deg:
                    pltpu.make_async_copy(onesb, deg_sh.at[dstg.at[p, u]],
                                          dsem[sw]).wait()

            @pl.when(j + LAG < nchunks)
            def _fire_next():
                u2 = (u + LAG) % GB
                p2 = lax.rem(g + (1 if u + LAG >= GB else 0), 2)
                pltpu.async_copy(x_hbm.at[srcg.at[p2, u2]], rowsb.at[sw],
                                 gsem[sw])

        def group(g, carry):
            chunk_work(g, 0)
            chunk_work(g, 1)

            @pl.when(g < ngroups - 1)
            def _prefetch():
                load_group(g + 1, lax.rem(g + 1, 2))
            for u in range(2, GB):
                chunk_work(g, u)
            return carry
        lax.fori_loop(0, ngroups, group, 0)

        for j in range(nchunks - LAS, nchunks):
            b = (j % GB) % RING
            pltpu.make_async_copy(rowsb.at[b], agg_sh.at[dstg.at[0, 0]],
                                  ssem[b]).wait()
            if compute_deg:
                pltpu.make_async_copy(onesb, deg_sh.at[dstg.at[0, 0]],
                                      dsem[b]).wait()

        plsc.subcore_barrier()

        if compute_deg:
            @pl.when(s < 10)
            def _deg_read():
                pltpu.sync_copy(deg_sh.at[pl.ds(s * DSL, DSL)], zd)
                pltpu.async_copy(zd, deg_out.at[pl.ds(c * N_pad + s * DSL, DSL)],
                                 dsem[0])

        for k in range(zrounds):
            ch = s + k * NS
            slot = k % 2
            if k >= 2:
                chp = s + (k - 2) * NS

                @pl.when(chp < nzch)
                def _wait_prev():
                    pltpu.make_async_copy(rowsb.at[slot],
                                          agg_out.at[c, pl.ds(chp * ZR, ZR)],
                                          ssem[slot]).wait()

            @pl.when(ch < nzch)
            def _stage():
                pltpu.async_copy(agg_sh.at[pl.ds(ch * ZR, ZR)], rowsb.at[slot],
                                 gsem[slot]).wait()
                pltpu.async_copy(rowsb.at[slot],
                                 agg_out.at[c, pl.ds(ch * ZR, ZR)], ssem[slot])

        for k in range(max(zrounds - 2, 0), zrounds):
            chl = s + k * NS
            slot = k % 2

            @pl.when(chl < nzch)
            def _drain_out():
                pltpu.make_async_copy(rowsb.at[slot],
                                      agg_out.at[c, pl.ds(chl * ZR, ZR)],
                                      ssem[slot]).wait()

        if compute_deg:
            @pl.when(s < 10)
            def _deg_drain():
                pltpu.make_async_copy(zd,
                                      deg_out.at[pl.ds(c * N_pad + s * DSL, DSL)],
                                      dsem[0]).wait()

    return sc_agg


@functools.lru_cache(maxsize=None)
def _make_tc_lin(N: int, D: int, BN: int):
    grid = (N // BN,)

    def body(x, w, b, out):
        out[...] = jnp.dot(x[...], w[...],
                           preferred_element_type=jnp.float32) + b[...]

    row = pl.BlockSpec((BN, D), lambda i: (i, 0))
    return pl.pallas_call(
        body,
        grid=grid,
        in_specs=[row, pl.BlockSpec((D, D), lambda i: (0, 0)),
                  pl.BlockSpec((1, D), lambda i: (0, 0))],
        out_specs=row,
        out_shape=jax.ShapeDtypeStruct((N, D), jnp.float32),
    )


@functools.lru_cache(maxsize=None)
def _make_tc1(N: int, N_pad: int, D: int, BN: int):
    grid = (N // BN,)

    def body(agg0, agg1, deg0, deg1, x, wl, wr, b, out):
        deg = jnp.maximum(deg0[0] + deg1[0], 1.0)
        mean = (agg0[0] + agg1[0]) / deg
        acc = jnp.dot(mean, wl[...], preferred_element_type=jnp.float32)
        acc += jnp.dot(x[...], wr[...], preferred_element_type=jnp.float32)
        out[...] = jnp.maximum(acc + b[...], 0.0)

    agg0s = pl.BlockSpec((1, BN, D), lambda i: (0, i, 0))
    agg1s = pl.BlockSpec((1, BN, D), lambda i: (1, i, 0))
    deg0s = pl.BlockSpec((1, BN, 1), lambda i: (0, i, 0))
    deg1s = pl.BlockSpec((1, BN, 1), lambda i: (1, i, 0))
    row = pl.BlockSpec((BN, D), lambda i: (i, 0))
    full = pl.BlockSpec((D, D), lambda i: (0, 0))
    bias = pl.BlockSpec((1, D), lambda i: (0, 0))
    return pl.pallas_call(
        body,
        grid=grid,
        in_specs=[agg0s, agg1s, deg0s, deg1s, row, full, full, bias],
        out_specs=row,
        out_shape=jax.ShapeDtypeStruct((N, D), jnp.float32),
    )


@functools.lru_cache(maxsize=None)
def _make_tc2(N: int, N_pad: int, D: int, C1: int, C2: int, BN: int):
    grid = (N // BN,)

    def body(agg0, agg1, deg0, deg1, h, wl, wr, b, wh1, bh1, wh2, bh2,
             h2_out, o1_out, o2_out):
        deg = jnp.maximum(deg0[0] + deg1[0], 1.0)
        mean = (agg0[0] + agg1[0]) / deg
        acc = jnp.dot(mean, wl[...], preferred_element_type=jnp.float32)
        acc += jnp.dot(h[...], wr[...], preferred_element_type=jnp.float32)
        h2 = acc + b[...]
        h2_out[...] = h2
        o1_out[...] = jnp.dot(h2, wh1[...], preferred_element_type=jnp.float32) + bh1[...]
        o2_out[...] = jnp.dot(h2, wh2[...], preferred_element_type=jnp.float32) + bh2[...]

    agg0s = pl.BlockSpec((1, BN, D), lambda i: (0, i, 0))
    agg1s = pl.BlockSpec((1, BN, D), lambda i: (1, i, 0))
    deg0s = pl.BlockSpec((1, BN, 1), lambda i: (0, i, 0))
    deg1s = pl.BlockSpec((1, BN, 1), lambda i: (1, i, 0))
    row = pl.BlockSpec((BN, D), lambda i: (i, 0))
    full = pl.BlockSpec((D, D), lambda i: (0, 0))
    return pl.pallas_call(
        body,
        grid=grid,
        in_specs=[agg0s, agg1s, deg0s, deg1s, row, full, full,
                  pl.BlockSpec((1, D), lambda i: (0, 0)),
                  pl.BlockSpec((D, C1), lambda i: (0, 0)),
                  pl.BlockSpec((1, C1), lambda i: (0, 0)),
                  pl.BlockSpec((D, C2), lambda i: (0, 0)),
                  pl.BlockSpec((1, C2), lambda i: (0, 0))],
        out_specs=[row,
                   pl.BlockSpec((BN, C1), lambda i: (i, 0)),
                   pl.BlockSpec((BN, C2), lambda i: (i, 0))],
        out_shape=[jax.ShapeDtypeStruct((N, D), jnp.float32),
                   jax.ShapeDtypeStruct((N, C1), jnp.float32),
                   jax.ShapeDtypeStruct((N, C2), jnp.float32)],
    )


def kernel(x, edge_index, W1l, b1l, W1r, W2l, b2l, W2r, Wh1, bh1, Wh2, bh2):
    N, D = x.shape
    E = edge_index.shape[1]
    C1 = Wh1.shape[0]
    C2 = Wh2.shape[0]
    BN = 2000 if N % 2000 == 0 else 8
    NW, NCH, E_pad, N_pad = _sc_geometry(N, E)

    ei = edge_index.astype(jnp.int32)
    npad = E_pad - E
    pad_src = (jnp.arange(npad, dtype=jnp.int32) * 37) % N
    pad_dst = N + (jnp.arange(npad, dtype=jnp.int32) % 128)
    src3 = jnp.concatenate([ei[0], pad_src]).reshape(NW, NCH, K)
    dst3 = jnp.concatenate([ei[1], pad_dst]).reshape(NW, NCH, K)

    tc1 = _make_tc1(N, N_pad, D, BN)
    tc2 = _make_tc2(N, N_pad, D, C1, C2, BN)

    aggp, degp = _make_sc_agg(N, D, E, True, 8)(x, src3, dst3)
    degp3 = degp.reshape(NC, N_pad, 1)
    h = tc1(aggp, aggp, degp3, degp3, x, W1l.T, W1r.T, b1l.reshape(1, D))

    (agg2p,) = _make_sc_agg(N, D, E, False, 16)(h, src3, dst3)
    h2, out1, out2 = tc2(agg2p, agg2p, degp3, degp3, h, W2l.T, W2r.T,
                         b2l.reshape(1, D),
                         Wh1.T, bh1.reshape(1, C1), Wh2.T, bh2.reshape(1, C2))
    return (out1, out2, h2)
